# baseline (speedup 1.0000x reference)
import functools

import jax
import jax.numpy as jnp
from jax import lax
from jax.experimental import pallas as pl
from jax.experimental.pallas import tpu as pltpu
from jax.experimental.pallas import tpu_sc as plsc

B = 16384
D = 32
L = 16
NC = 2
NS = 16
NW = NC * NS
V = 1000000
G = 512
G2 = 2 * G
MAIN = 999936
NCHUNK = MAIN // G
TAIL = V - MAIN
NJ0 = 62
NJW = 61
NJ = 62
JT = NJ
JX = NJ + 1
CAP = B + 64 * L

_SC_MESH = plsc.VectorSubcoreMesh(core_axis_name="c", subcore_axis_name="s",
                                  num_cores=NC, num_subcores=NS)
_PARAMS = pltpu.CompilerParams(needs_layout_passes=False)


def _iota():
    return lax.iota(jnp.int32, L)


def _bc(x, dtype=jnp.float32):
    return jnp.full((L,), x, dtype)


def _owner_j(idv):
    c = lax.shift_right_logical(idv, 9)
    w = jnp.where(c < NJ0, 0, (c - NJ0) // NJW + 1)
    j = jnp.where(c < NJ0, c, (c - NJ0) % NJW)
    return w, j


def _match_pass(ids_v, wid, matchb_v, matchid_v):
    def body(k, cnt):
        idv = ids_v[pl.ds(k * L, L)]
        bv = k * L + _iota()
        w, _ = _owner_j(idv)
        m = jnp.logical_and(w == wid, idv < MAIN)
        m = jnp.logical_or(
            m, jnp.logical_and(idv >= MAIN, _bc(wid, jnp.int32) == NW - 1))
        pos = cnt + plsc.cumsum(m.astype(jnp.int32)) - 1
        plsc.store_scatter(matchid_v, [pos], idv, mask=m)
        plsc.store_scatter(matchb_v, [pos], bv, mask=m)
        return cnt + plsc.all_reduce_population_count(m)[0]
    return lax.fori_loop(0, B // L, body, 0)


def _bucket_j(idv, cnt, lane):
    _, j = _owner_j(idv)
    j = jnp.where(idv >= MAIN, JT, j)
    return jnp.where(lane < cnt, j, JX)


def _bucket_sort(cnt, matchb_v, matchid_v, minib_v, minioff_v,
                 hist_v, starts_v):
    ones = _bc(1, jnp.int32)
    zeros = jnp.zeros((L,), jnp.int32)
    nloops = (cnt + L - 1) // L
    for k in range(4):
        hist_v[pl.ds(k * L, L)] = zeros

    def hbody(k, carry):
        lane = k * L + _iota()
        idv = matchid_v[pl.ds(k * L, L)]
        jv = _bucket_j(idv, cnt, lane)
        for l in range(L):
            plsc.addupdate_scatter(hist_v, [_bc(jv[l], jnp.int32)], ones,
                                   mask=_iota() == l)
        return carry
    lax.fori_loop(0, nloops, hbody, 0)

    def pbody(k, carry):
        h = hist_v[pl.ds(k * L, L)]
        sal = jnp.bitwise_and(h + (L - 1), _bc(~(L - 1), jnp.int32))
        inc = plsc.cumsum(sal)
        starts_v[pl.ds(k * L, L)] = carry + inc - sal
        return carry + inc[L - 1]
    total = lax.fori_loop(0, 4, pbody, 0)

    neg = _bc(-1, jnp.int32)

    def cbody(k, carry):
        minib_v[pl.ds(k * L, L)] = neg
        return carry
    lax.fori_loop(0, (total + L - 1) // L, cbody, 0)

    def rbody(k, carry):
        hist_v[pl.ds(k * L, L)] = starts_v[pl.ds(k * L, L)]
        return carry
    lax.fori_loop(0, 4, rbody, 0)

    def sbody(k, carry):
        lane = k * L + _iota()
        idv = matchid_v[pl.ds(k * L, L)]
        bv = matchb_v[pl.ds(k * L, L)]
        jv = _bucket_j(idv, cnt, lane)
        for l in range(L):
            lm = _iota() == l
            jl = _bc(jv[l], jnp.int32)
            posl = _bc(plsc.load_gather(hist_v, [jl])[0], jnp.int32)
            plsc.store_scatter(minioff_v, [posl], idv, mask=lm)
            plsc.store_scatter(minib_v, [posl], bv, mask=lm)
            plsc.addupdate_scatter(hist_v, [jl], ones, mask=lm)
        return carry
    lax.fori_loop(0, nloops, sbody, 0)


@functools.partial(
    pl.kernel,
    out_type=jax.ShapeDtypeStruct((B, 128), jnp.float32),
    mesh=_SC_MESH,
    compiler_params=_PARAMS,
    scratch_types=[
        pltpu.VMEM((B,), jnp.int32),
        pltpu.VMEM((B,), jnp.int32),
        pltpu.VMEM((B,), jnp.int32),
        pltpu.VMEM((CAP,), jnp.int32),
        pltpu.VMEM((CAP,), jnp.int32),
        pltpu.VMEM((64,), jnp.int32),
        pltpu.VMEM((64,), jnp.int32),
        pltpu.VMEM((D, G2), jnp.float32),
        pltpu.VMEM((L, 128), jnp.float32),
        pltpu.VMEM((L,), jnp.int32),
        pltpu.SemaphoreType.DMA,
        pltpu.SemaphoreType.DMA,
        pltpu.SemaphoreType.DMA,
        pltpu.SemaphoreType.DMA,
        pltpu.SemaphoreType.DMA,
    ],
)
def _scan(ids_hbm, t_hbm, rows_hbm, ids_v, matchb_v, matchid_v,
          minib_v, minioff_v, hist_v, starts_v, slab_v,
          rows_v, bidx_v, s0, s1, s2, s3, sem2):
    wid = lax.axis_index("s") * NC + lax.axis_index("c")
    pltpu.sync_copy(ids_hbm, ids_v)
    cnt = _match_pass(ids_v, wid, matchb_v, matchid_v)
    _bucket_sort(cnt, matchb_v, matchid_v, minib_v, minioff_v,
                 hist_v, starts_v)

    dlo = _iota()
    dhi = dlo + L
    qsems = (s0, s1, s2, s3)
    cstart = jnp.where(wid == 0, 0, NJ0 + (wid - 1) * NJW)
    njw = jnp.where(wid == 0, NJ0, NJW)

    def issue(c0, width):
        for q in range(4):
            pltpu.async_copy(
                t_hbm.at[pl.ds(q * 8, 8),
                         pl.ds(pl.multiple_of(c0 * G, 128), width)],
                slab_v.at[pl.ds(q * 8, 8), pl.ds(0, width)], qsems[q])

    def wait_slab(c0, width):
        for q in range(4):
            pltpu.make_async_copy(
                t_hbm.at[pl.ds(q * 8, 8),
                         pl.ds(pl.multiple_of(c0 * G, 128), width)],
                slab_v.at[pl.ds(q * 8, 8), pl.ds(0, width)],
                qsems[q]).wait()

    def emit_groups(j, lo, soff):
        gstart = plsc.load_gather(starts_v, [_bc(j, jnp.int32)])[0]
        gend = plsc.load_gather(starts_v, [_bc(j + 1, jnp.int32)])[0]

        def gbody(g, carry):
            sl = pl.ds(gstart + g * L, L)
            bv = minib_v[sl]
            idvb = minioff_v[sl]
            valid = bv >= 0
            offv = jnp.where(valid, idvb - lo + soff, 0)
            bidx_v[...] = jnp.where(valid, bv, -1)
            for l in range(L):
                off = _bc(offv[l], jnp.int32)
                rows_v[l, pl.ds(0, L)] = plsc.load_gather(slab_v, [dlo, off])
                rows_v[l, pl.ds(L, L)] = plsc.load_gather(slab_v, [dhi, off])
            pltpu.async_copy(
                rows_v, rows_hbm.at[plsc.Indices(bidx_v, ignored_value=-1)],
                sem2).wait()
            return carry
        lax.fori_loop(0, lax.shift_right_logical(gend - gstart, 4), gbody, 0)

    def pair_body(jp, carry):
        j0 = jp * 2
        j1 = jp * 2 + 1
        c0 = cstart + j0

        @pl.when(j1 < njw)
        def _():
            issue(c0, G2)
            wait_slab(c0, G2)
            emit_groups(j0, c0 * G, 0)
            emit_groups(j1, (c0 + 1) * G, G)

        @pl.when(jnp.logical_and(j0 < njw, j1 >= njw))
        def _():
            issue(c0, G)
            wait_slab(c0, G)
            emit_groups(j0, c0 * G, 0)
        return carry

    lax.fori_loop(0, NJ0 // 2, pair_body, 0)

    @pl.when(wid == NW - 1)
    def _():
        for dd in range(D):
            pltpu.sync_copy(t_hbm.at[dd, pl.ds(MAIN, TAIL)],
                            slab_v.at[dd, pl.ds(0, TAIL)])
        emit_groups(JT, MAIN, 0)


BPW = B // NW
P = 128


@functools.partial(
    pl.kernel,
    out_type=jax.ShapeDtypeStruct((B,), jnp.float32),
    mesh=_SC_MESH,
    compiler_params=_PARAMS,
    scratch_types=[
        pltpu.VMEM((BPW,), jnp.int32),
        pltpu.VMEM((BPW,), jnp.int32),
        pltpu.VMEM((BPW,), jnp.float32),
        pltpu.VMEM((BPW,), jnp.float32),
        pltpu.VMEM((L,), jnp.float32),
        pltpu.VMEM((P, 128), jnp.float32),
        pltpu.VMEM((P, 128), jnp.float32),
        pltpu.VMEM((BPW,), jnp.float32),
        pltpu.SemaphoreType.DMA,
        pltpu.SemaphoreType.DMA,
        pltpu.SemaphoreType.DMA,
        pltpu.SemaphoreType.DMA,
    ],
)
def _finish(uid_hbm, iid_hbm, urows_hbm, irows_hbm, ub_hbm, ib_hbm, gb_hbm,
            out_hbm, uid_v, iid_v, ubv_v, ibv_v, gb_v, up_v, ip_v, out_v,
            semu, semi, semub, semib):
    wid = lax.axis_index("s") * NC + lax.axis_index("c")
    b0 = wid * BPW
    pltpu.sync_copy(uid_hbm.at[pl.ds(b0, BPW)], uid_v)
    pltpu.sync_copy(iid_hbm.at[pl.ds(b0, BPW)], iid_v)
    pltpu.sync_copy(gb_hbm, gb_v)
    cub = pltpu.async_copy(ub_hbm.at[plsc.Indices(uid_v)], ubv_v, semub)
    cib = pltpu.async_copy(ib_hbm.at[plsc.Indices(iid_v)], ibv_v, semib)
    cub.wait()
    cib.wait()
    gb = gb_v[...]

    def piece(p, carry):
        pb = b0 + p * P
        cu = pltpu.async_copy(urows_hbm.at[pl.ds(pb, P), :], up_v, semu)
        ci = pltpu.async_copy(irows_hbm.at[pl.ds(pb, P), :], ip_v, semi)
        cu.wait()
        ci.wait()

        def group(g, carry2):
            row = g * L + _iota()
            acc = (ubv_v[pl.ds(p * P + g * L, L)]
                   + ibv_v[pl.ds(p * P + g * L, L)] + gb)
            for d in range(D):
                col = _bc(d, jnp.int32)
                u = plsc.load_gather(up_v, [row, col])
                it = plsc.load_gather(ip_v, [row, col])
                acc = acc + u * it
            out_v[pl.ds(p * P + g * L, L)] = acc
            return carry2
        lax.fori_loop(0, P // L, group, 0)
        return carry

    lax.fori_loop(0, BPW // P, piece, 0)
    pltpu.sync_copy(out_v, out_hbm.at[pl.ds(b0, BPW)])


def kernel(user_ids, item_ids, user_table, item_table, user_bias_table,
           item_bias_table, global_bias):
    gb16 = jnp.broadcast_to(global_bias.astype(jnp.float32), (L,))
    u_rows = _scan(user_ids, user_table.T)
    i_rows = _scan(item_ids, item_table.T)
    return _finish(user_ids, item_ids, u_rows, i_rows,
                   user_bias_table.reshape(-1), item_bias_table.reshape(-1),
                   gb16)
